# Initial kernel scaffold; baseline (speedup 1.0000x reference)
import jax
import jax.numpy as jnp
from jax.experimental import pallas as pl


def kernel(x, edge_index, edge_attr, batch, W_node, b_node, Wf1, bf1, Ws1, bs1, Wf2, bf2, Ws2, bs2, W_fc1, b_fc1, W_fc2, b_fc2):
    raise NotImplementedError("write your pallas kernel here")


# speedup vs baseline: 3.2914x; 3.2914x over previous
import functools

import jax
import jax.numpy as jnp
from jax import lax
from jax.experimental import pallas as pl
from jax.experimental.pallas import tpu as pltpu
from jax.experimental.pallas import tpu_sc as plsc

_N = 10000
_E = 320000
_D = 128
_DE = 16
_G = 64

_NC = 2
_NS = 16
_NW = _NC * _NS
_EW = _E // _NW
_CH = 80
_NCH = _EW // _CH
_NPAIR = _NCH // 2
_NPAD = 10240
_NSL = _NPAD // _NS

_SCH = 40
_SNCH = _EW // _SCH
_SNPAIR = _SNCH // 2
_OBR = 40
_NOB = _NSL // _OBR

_f32 = jnp.float32


def _lrelu(t):
    return jnp.maximum(t, 0.01 * t)


def _sc_gather_body(h_hbm, dsti_hbm, srci_hbm, pi_hbm, pj_hbm,
                    idxd, idxs, bi0, bi1, bj0, bj1, gs0, gs1, os0, os1):
    cid = lax.axis_index("c")
    sid = lax.axis_index("s")
    wid = sid * _NC + cid
    ebase = wid * _EW

    pltpu.sync_copy(dsti_hbm.at[wid], idxd)
    pltpu.sync_copy(srci_hbm.at[wid], idxs)

    bi = (bi0, bi1)
    bj = (bj0, bj1)
    gsem = (gs0, gs1)
    osem = (os0, os1)

    def gathers(ch, b):
        g1 = pltpu.async_copy(h_hbm.at[idxd.at[ch]], bi[b], gsem[b])
        g2 = pltpu.async_copy(h_hbm.at[idxs.at[ch]], bj[b], gsem[b])
        return g1, g2

    def outs(ch, b):
        off = ebase + ch * _CH
        pltpu.async_copy(bi[b], pi_hbm.at[pl.ds(off, _CH)], osem[b])
        pltpu.async_copy(bj[b], pj_hbm.at[pl.ds(off, _CH)], osem[b])

    def drain_outs(b):
        pltpu.make_async_copy(bi[b], pi_hbm.at[pl.ds(0, _CH)], osem[b]).wait()
        pltpu.make_async_copy(bj[b], pj_hbm.at[pl.ds(0, _CH)], osem[b]).wait()

    def pair(p, _):
        base = 2 * p

        @pl.when(p > 0)
        def _():
            drain_outs(0)
            drain_outs(1)

        g0a, g0b = gathers(base, 0)
        g1a, g1b = gathers(base + 1, 1)
        g0a.wait()
        g0b.wait()
        outs(base, 0)
        g1a.wait()
        g1b.wait()
        outs(base + 1, 1)
        return ()

    lax.fori_loop(0, _NPAIR, pair, (), unroll=False)

    drain_outs(0)
    ta, tb = gathers(_NCH - 1, 0)
    ta.wait()
    tb.wait()
    outs(_NCH - 1, 0)
    drain_outs(1)
    drain_outs(0)


def _sc_scatter_body(msg_hbm, dsti_hbm, zeros_hbm, out_hbm,
                     agg, idxd, mb0, mb1, ob, ls0, ls1):
    cid = lax.axis_index("c")
    sid = lax.axis_index("s")
    wid = sid * _NC + cid
    ebase = wid * _EW
    rbase = sid * _NSL

    pltpu.sync_copy(zeros_hbm, ob)
    for j in range(_NOB):
        pltpu.sync_copy(ob, agg.at[pl.ds(rbase + j * _OBR, _OBR)])
    pltpu.sync_copy(dsti_hbm.at[wid], idxd)
    plsc.subcore_barrier()

    mb = (mb0, mb1)
    lsem = (ls0, ls1)

    def load(ch, b):
        off = ebase + ch * _SCH
        pltpu.async_copy(msg_hbm.at[pl.ds(off, _SCH)], mb[b], lsem[b])

    def wait_load(b):
        pltpu.make_async_copy(msg_hbm.at[pl.ds(0, _SCH)], mb[b],
                              lsem[b]).wait()

    load(0, 0)
    load(1, 1)

    def pair(p, _):
        for b in range(2):
            ch = 2 * p + b
            wait_load(b)
            pltpu.sync_copy(mb[b], agg.at[idxd.at[ch]], add=True)

            @pl.when(ch < _SNCH - 2)
            def _():
                load(ch + 2, b)

        return ()

    lax.fori_loop(0, _SNPAIR, pair, (), unroll=False)

    plsc.subcore_barrier()

    for j in range(_NOB):
        sl = pl.ds(rbase + j * _OBR, _OBR)
        pltpu.sync_copy(agg.at[sl], ob)
        pltpu.sync_copy(ob, out_hbm.at[cid].at[sl])


@functools.cache
def _sc_gather():
    return pl.kernel(
        _sc_gather_body,
        mesh=plsc.VectorSubcoreMesh(core_axis_name="c", subcore_axis_name="s"),
        out_type=(
            jax.ShapeDtypeStruct((_E, _D), _f32),
            jax.ShapeDtypeStruct((_E, _D), _f32),
        ),
        scratch_types=(
            pltpu.VMEM((_NCH, _CH), jnp.int32),
            pltpu.VMEM((_NCH, _CH), jnp.int32),
            pltpu.VMEM((_CH, _D), _f32),
            pltpu.VMEM((_CH, _D), _f32),
            pltpu.VMEM((_CH, _D), _f32),
            pltpu.VMEM((_CH, _D), _f32),
            pltpu.SemaphoreType.DMA,
            pltpu.SemaphoreType.DMA,
            pltpu.SemaphoreType.DMA,
            pltpu.SemaphoreType.DMA,
        ),
    )


@functools.cache
def _sc_scatter():
    return pl.kernel(
        _sc_scatter_body,
        mesh=plsc.VectorSubcoreMesh(core_axis_name="c", subcore_axis_name="s"),
        out_type=jax.ShapeDtypeStruct((_NC, _NPAD, _D), _f32),
        scratch_types=(
            pltpu.VMEM_SHARED((_NPAD, _D), _f32),
            pltpu.VMEM((_SNCH, _SCH), jnp.int32),
            pltpu.VMEM((_SCH, _D), _f32),
            pltpu.VMEM((_SCH, _D), _f32),
            pltpu.VMEM((_OBR, _D), _f32),
            pltpu.SemaphoreType.DMA,
            pltpu.SemaphoreType.DMA,
        ),
    )


_BN = 2000
_BE = 2000


def _tc_transform_body(x_ref, w_ref, b_ref, h_ref):
    h_ref[...] = _lrelu(
        jnp.dot(x_ref[...], w_ref[...], preferred_element_type=_f32)
        + b_ref[...])


def _tc_transform(x, w, b):
    return pl.pallas_call(
        _tc_transform_body,
        grid=(_N // _BN,),
        in_specs=[
            pl.BlockSpec((_BN, _D), lambda i: (i, 0)),
            pl.BlockSpec((_D, _D), lambda i: (0, 0)),
            pl.BlockSpec((1, _D), lambda i: (0, 0)),
        ],
        out_specs=pl.BlockSpec((_BN, _D), lambda i: (i, 0)),
        out_shape=jax.ShapeDtypeStruct((_N, _D), _f32),
    )(x, w, b)


def _tc_msg_body(pi_ref, pj_ref, ea_ref, wfi_ref, wfj_ref, wfe_ref, bf_ref,
                 wsi_ref, wsj_ref, wse_ref, bs_ref, msg_ref):
    pi = pi_ref[...]
    pj = pj_ref[...]
    ea = ea_ref[...]
    f = (jnp.dot(pi, wfi_ref[...], preferred_element_type=_f32)
         + jnp.dot(pj, wfj_ref[...], preferred_element_type=_f32)
         + jnp.dot(ea, wfe_ref[...], preferred_element_type=_f32)
         + bf_ref[...])
    s = (jnp.dot(pi, wsi_ref[...], preferred_element_type=_f32)
         + jnp.dot(pj, wsj_ref[...], preferred_element_type=_f32)
         + jnp.dot(ea, wse_ref[...], preferred_element_type=_f32)
         + bs_ref[...])
    msg_ref[...] = jax.nn.sigmoid(f) * jax.nn.softplus(s)


def _tc_msg(pi, pj, ea, wfi, wfj, wfe, bf, wsi, wsj, wse, bs):
    full = lambda r, c: pl.BlockSpec((r, c), lambda i: (0, 0))
    return pl.pallas_call(
        _tc_msg_body,
        grid=(_E // _BE,),
        in_specs=[
            pl.BlockSpec((_BE, _D), lambda i: (i, 0)),
            pl.BlockSpec((_BE, _D), lambda i: (i, 0)),
            pl.BlockSpec((_BE, _DE), lambda i: (i, 0)),
            full(_D, _D), full(_D, _D), full(_DE, _D), full(1, _D),
            full(_D, _D), full(_D, _D), full(_DE, _D), full(1, _D),
        ],
        out_specs=pl.BlockSpec((_BE, _D), lambda i: (i, 0)),
        out_shape=jax.ShapeDtypeStruct((_E, _D), _f32),
    )(pi, pj, ea, wfi, wfj, wfe, bf, wsi, wsj, wse, bs)


def _tc_update_body(h_ref, p0_ref, p1_ref, o_ref):
    o_ref[...] = _lrelu(h_ref[...] + p0_ref[...] + p1_ref[...])


def _tc_update(h, p0, p1):
    return pl.pallas_call(
        _tc_update_body,
        grid=(_N // _BN,),
        in_specs=[pl.BlockSpec((_BN, _D), lambda i: (i, 0))] * 3,
        out_specs=pl.BlockSpec((_BN, _D), lambda i: (i, 0)),
        out_shape=jax.ShapeDtypeStruct((_N, _D), _f32),
    )(h, p0, p1)


def _tc_head_body(h_ref, p0_ref, p1_ref, batch_ref, w1_ref, b1_ref,
                  w2_ref, b2_ref, atom_ref, out_ref, pooled):
    i = pl.program_id(0)
    ae = _lrelu(h_ref[...] + p0_ref[...] + p1_ref[...])
    atom_ref[...] = ae
    ids = batch_ref[0, 0, :]
    onehot = (lax.broadcasted_iota(jnp.int32, (_G, _BN), 0)
              == ids[None, :]).astype(_f32)
    contrib = jnp.dot(onehot, ae, preferred_element_type=_f32)

    @pl.when(i == 0)
    def _():
        pooled[...] = contrib

    @pl.when(i > 0)
    def _():
        pooled[...] += contrib

    @pl.when(i == _N // _BN - 1)
    def _():
        p = pooled[...]
        nrm = jnp.sqrt(jnp.sum(p * p, axis=1, keepdims=True))
        p = p / jnp.maximum(nrm, 1e-12)
        h2 = _lrelu(jnp.dot(p, w1_ref[...], preferred_element_type=_f32)
                    + b1_ref[...])
        out_ref[...] = (jnp.dot(h2, w2_ref[...], preferred_element_type=_f32)
                        + b2_ref[...])


def _tc_head(h, p0, p1, batch3d, w1, b1, w2, b2):
    full = lambda r, c: pl.BlockSpec((r, c), lambda i: (0, 0))
    return pl.pallas_call(
        _tc_head_body,
        grid=(_N // _BN,),
        in_specs=[
            pl.BlockSpec((_BN, _D), lambda i: (i, 0)),
            pl.BlockSpec((_BN, _D), lambda i: (i, 0)),
            pl.BlockSpec((_BN, _D), lambda i: (i, 0)),
            pl.BlockSpec((1, 1, _BN), lambda i: (i, 0, 0)),
            full(_D, _D), full(1, _D), full(_D, 1), full(1, 1),
        ],
        out_specs=[
            pl.BlockSpec((_BN, _D), lambda i: (i, 0)),
            pl.BlockSpec((_G, 1), lambda i: (0, 0)),
        ],
        out_shape=[
            jax.ShapeDtypeStruct((_N, _D), _f32),
            jax.ShapeDtypeStruct((_G, 1), _f32),
        ],
        scratch_shapes=[pltpu.VMEM((_G, _D), _f32)],
    )(h, p0, p1, batch3d, w1, b1, w2, b2)


def _layer(h, dst3d_g, src3d_g, dst3d_s, ea, zeros_sl, Wf, bf, Ws, bs):
    pi, pj = _sc_gather()(h, dst3d_g, src3d_g)
    msg = _tc_msg(
        pi, pj, ea,
        Wf[:_D], Wf[_D:2 * _D], Wf[2 * _D:], bf.reshape(1, _D),
        Ws[:_D], Ws[_D:2 * _D], Ws[2 * _D:], bs.reshape(1, _D),
    )
    parts = _sc_scatter()(msg, dst3d_s, zeros_sl)
    return parts[0], parts[1]


def kernel(x, edge_index, edge_attr, batch, W_node, b_node, Wf1, bf1, Ws1, bs1,
           Wf2, bf2, Ws2, bs2, W_fc1, b_fc1, W_fc2, b_fc2):
    dsti = edge_index[1].astype(jnp.int32)
    srci = edge_index[0].astype(jnp.int32)
    dst3d_g = dsti.reshape(_NW, _NCH, _CH)
    src3d_g = srci.reshape(_NW, _NCH, _CH)
    dst3d_s = dsti.reshape(_NW, _SNCH, _SCH)
    batch3d = batch.astype(jnp.int32).reshape(_N // _BN, 1, _BN)
    zeros_sl = jnp.zeros((_OBR, _D), _f32)

    h = _tc_transform(x, W_node, b_node.reshape(1, _D))
    p0, p1 = _layer(h, dst3d_g, src3d_g, dst3d_s, edge_attr, zeros_sl,
                    Wf1, bf1, Ws1, bs1)
    h1 = _tc_update(h, p0, p1)
    q0, q1 = _layer(h1, dst3d_g, src3d_g, dst3d_s, edge_attr, zeros_sl,
                    Wf2, bf2, Ws2, bs2)
    atom_embs, out = _tc_head(h1, q0, q1, batch3d,
                              W_fc1, b_fc1.reshape(1, _D),
                              W_fc2, b_fc2.reshape(1, 1))
    return (out, atom_embs)


# speedup vs baseline: 3.2937x; 1.0007x over previous
import functools

import jax
import jax.numpy as jnp
from jax import lax
from jax.experimental import pallas as pl
from jax.experimental.pallas import tpu as pltpu
from jax.experimental.pallas import tpu_sc as plsc

_N = 10000
_E = 320000
_D = 128
_DE = 16
_G = 64

_NC = 2
_NS = 16
_NW = _NC * _NS
_EW = _E // _NW
_CH = 80
_NCH = _EW // _CH
_NPAIR = _NCH // 2
_NPAD = 10240
_NSL = _NPAD // _NS

_SCH = 40
_SNCH = _EW // _SCH
_SNPAIR = _SNCH // 2
_OBR = 40
_NOB = _NSL // _OBR

_f32 = jnp.float32
_bf16 = jnp.bfloat16


def _lrelu(t):
    return jnp.maximum(t, 0.01 * t)


def _sc_gather_body(h_hbm, dsti_hbm, srci_hbm, pi_hbm, pj_hbm,
                    idxd, idxs, bi0, bi1, bj0, bj1, gs0, gs1, os0, os1):
    cid = lax.axis_index("c")
    sid = lax.axis_index("s")
    wid = sid * _NC + cid
    ebase = wid * _EW

    pltpu.sync_copy(dsti_hbm.at[wid], idxd)
    pltpu.sync_copy(srci_hbm.at[wid], idxs)

    bi = (bi0, bi1)
    bj = (bj0, bj1)
    gsem = (gs0, gs1)
    osem = (os0, os1)

    def gathers(ch, b):
        g1 = pltpu.async_copy(h_hbm.at[idxd.at[ch]], bi[b], gsem[b])
        g2 = pltpu.async_copy(h_hbm.at[idxs.at[ch]], bj[b], gsem[b])
        return g1, g2

    def outs(ch, b):
        off = ebase + ch * _CH
        pltpu.async_copy(bi[b], pi_hbm.at[pl.ds(off, _CH)], osem[b])
        pltpu.async_copy(bj[b], pj_hbm.at[pl.ds(off, _CH)], osem[b])

    def drain_outs(b):
        pltpu.make_async_copy(bi[b], pi_hbm.at[pl.ds(0, _CH)], osem[b]).wait()
        pltpu.make_async_copy(bj[b], pj_hbm.at[pl.ds(0, _CH)], osem[b]).wait()

    def pair(p, _):
        base = 2 * p

        @pl.when(p > 0)
        def _():
            drain_outs(0)
            drain_outs(1)

        g0a, g0b = gathers(base, 0)
        g1a, g1b = gathers(base + 1, 1)
        g0a.wait()
        g0b.wait()
        outs(base, 0)
        g1a.wait()
        g1b.wait()
        outs(base + 1, 1)
        return ()

    lax.fori_loop(0, _NPAIR, pair, (), unroll=False)

    drain_outs(0)
    ta, tb = gathers(_NCH - 1, 0)
    ta.wait()
    tb.wait()
    outs(_NCH - 1, 0)
    drain_outs(1)
    drain_outs(0)


def _sc_scatter_body(msg_hbm, dsti_hbm, zeros_hbm, out_hbm,
                     agg, idxd, mb0, mb1, ob, ls0, ls1):
    cid = lax.axis_index("c")
    sid = lax.axis_index("s")
    wid = sid * _NC + cid
    ebase = wid * _EW
    rbase = sid * _NSL

    pltpu.sync_copy(zeros_hbm, ob)
    for j in range(_NOB):
        pltpu.sync_copy(ob, agg.at[pl.ds(rbase + j * _OBR, _OBR)])
    pltpu.sync_copy(dsti_hbm.at[wid], idxd)
    plsc.subcore_barrier()

    mb = (mb0, mb1)
    lsem = (ls0, ls1)

    def load(ch, b):
        off = ebase + ch * _SCH
        pltpu.async_copy(msg_hbm.at[pl.ds(off, _SCH)], mb[b], lsem[b])

    def wait_load(b):
        pltpu.make_async_copy(msg_hbm.at[pl.ds(0, _SCH)], mb[b],
                              lsem[b]).wait()

    load(0, 0)
    load(1, 1)

    def pair(p, _):
        for b in range(2):
            ch = 2 * p + b
            wait_load(b)
            pltpu.sync_copy(mb[b], agg.at[idxd.at[ch]], add=True)

            @pl.when(ch < _SNCH - 2)
            def _():
                load(ch + 2, b)

        return ()

    lax.fori_loop(0, _SNPAIR, pair, (), unroll=False)

    plsc.subcore_barrier()

    for j in range(_NOB):
        sl = pl.ds(rbase + j * _OBR, _OBR)
        pltpu.sync_copy(agg.at[sl], ob)
        pltpu.sync_copy(ob, out_hbm.at[cid].at[sl])


@functools.cache
def _sc_gather():
    return pl.kernel(
        _sc_gather_body,
        mesh=plsc.VectorSubcoreMesh(core_axis_name="c", subcore_axis_name="s"),
        out_type=(
            jax.ShapeDtypeStruct((_E, _D), _f32),
            jax.ShapeDtypeStruct((_E, _D), _f32),
        ),
        scratch_types=(
            pltpu.VMEM((_NCH, _CH), jnp.int32),
            pltpu.VMEM((_NCH, _CH), jnp.int32),
            pltpu.VMEM((_CH, _D), _f32),
            pltpu.VMEM((_CH, _D), _f32),
            pltpu.VMEM((_CH, _D), _f32),
            pltpu.VMEM((_CH, _D), _f32),
            pltpu.SemaphoreType.DMA,
            pltpu.SemaphoreType.DMA,
            pltpu.SemaphoreType.DMA,
            pltpu.SemaphoreType.DMA,
        ),
    )


@functools.cache
def _sc_scatter():
    return pl.kernel(
        _sc_scatter_body,
        mesh=plsc.VectorSubcoreMesh(core_axis_name="c", subcore_axis_name="s"),
        out_type=jax.ShapeDtypeStruct((_NC, _NPAD, _D), _f32),
        scratch_types=(
            pltpu.VMEM_SHARED((_NPAD, _D), _f32),
            pltpu.VMEM((_SNCH, _SCH), jnp.int32),
            pltpu.VMEM((_SCH, _D), _f32),
            pltpu.VMEM((_SCH, _D), _f32),
            pltpu.VMEM((_OBR, _D), _f32),
            pltpu.SemaphoreType.DMA,
            pltpu.SemaphoreType.DMA,
        ),
    )


_BN = 2000
_BE = 2000


def _tc_transform_body(x_ref, w_ref, b_ref, h_ref):
    h_ref[...] = _lrelu(
        jnp.dot(x_ref[...], w_ref[...], preferred_element_type=_f32)
        + b_ref[...])


def _tc_transform(x, w, b):
    return pl.pallas_call(
        _tc_transform_body,
        grid=(_N // _BN,),
        in_specs=[
            pl.BlockSpec((_BN, _D), lambda i: (i, 0)),
            pl.BlockSpec((_D, _D), lambda i: (0, 0)),
            pl.BlockSpec((1, _D), lambda i: (0, 0)),
        ],
        out_specs=pl.BlockSpec((_BN, _D), lambda i: (i, 0)),
        out_shape=jax.ShapeDtypeStruct((_N, _D), _f32),
    )(x, w, b)


def _tc_msg_body(pi_ref, pj_ref, ea_ref, wfi_ref, wfj_ref, wfe_ref, bf_ref,
                 wsi_ref, wsj_ref, wse_ref, bs_ref, msg_ref):
    pi = pi_ref[...].astype(_bf16)
    pj = pj_ref[...].astype(_bf16)
    ea = ea_ref[...]
    f = (jnp.dot(pi, wfi_ref[...], preferred_element_type=_f32)
         + jnp.dot(pj, wfj_ref[...], preferred_element_type=_f32)
         + jnp.dot(ea, wfe_ref[...], preferred_element_type=_f32)
         + bf_ref[...])
    s = (jnp.dot(pi, wsi_ref[...], preferred_element_type=_f32)
         + jnp.dot(pj, wsj_ref[...], preferred_element_type=_f32)
         + jnp.dot(ea, wse_ref[...], preferred_element_type=_f32)
         + bs_ref[...])
    msg_ref[...] = jax.nn.sigmoid(f) * jax.nn.softplus(s)


def _tc_msg(pi, pj, ea, wfi, wfj, wfe, bf, wsi, wsj, wse, bs):
    full = lambda r, c: pl.BlockSpec((r, c), lambda i: (0, 0))
    return pl.pallas_call(
        _tc_msg_body,
        grid=(_E // _BE,),
        in_specs=[
            pl.BlockSpec((_BE, _D), lambda i: (i, 0)),
            pl.BlockSpec((_BE, _D), lambda i: (i, 0)),
            pl.BlockSpec((_BE, _DE), lambda i: (i, 0)),
            full(_D, _D), full(_D, _D), full(_DE, _D), full(1, _D),
            full(_D, _D), full(_D, _D), full(_DE, _D), full(1, _D),
        ],
        out_specs=pl.BlockSpec((_BE, _D), lambda i: (i, 0)),
        out_shape=jax.ShapeDtypeStruct((_E, _D), _f32),
    )(pi, pj, ea, wfi, wfj, wfe, bf, wsi, wsj, wse, bs)


def _tc_update_body(h_ref, p0_ref, p1_ref, o_ref):
    o_ref[...] = _lrelu(h_ref[...] + p0_ref[...] + p1_ref[...])


def _tc_update(h, p0, p1):
    return pl.pallas_call(
        _tc_update_body,
        grid=(_N // _BN,),
        in_specs=[pl.BlockSpec((_BN, _D), lambda i: (i, 0))] * 3,
        out_specs=pl.BlockSpec((_BN, _D), lambda i: (i, 0)),
        out_shape=jax.ShapeDtypeStruct((_N, _D), _f32),
    )(h, p0, p1)


def _tc_head_body(h_ref, p0_ref, p1_ref, batch_ref, w1_ref, b1_ref,
                  w2_ref, b2_ref, atom_ref, out_ref, pooled):
    i = pl.program_id(0)
    ae = _lrelu(h_ref[...] + p0_ref[...] + p1_ref[...])
    atom_ref[...] = ae
    ids = batch_ref[0, 0, :]
    onehot = (lax.broadcasted_iota(jnp.int32, (_G, _BN), 0)
              == ids[None, :]).astype(_f32)
    contrib = jnp.dot(onehot, ae, preferred_element_type=_f32)

    @pl.when(i == 0)
    def _():
        pooled[...] = contrib

    @pl.when(i > 0)
    def _():
        pooled[...] += contrib

    @pl.when(i == _N // _BN - 1)
    def _():
        p = pooled[...]
        nrm = jnp.sqrt(jnp.sum(p * p, axis=1, keepdims=True))
        p = p / jnp.maximum(nrm, 1e-12)
        h2 = _lrelu(jnp.dot(p, w1_ref[...], preferred_element_type=_f32)
                    + b1_ref[...])
        out_ref[...] = (jnp.dot(h2, w2_ref[...], preferred_element_type=_f32)
                        + b2_ref[...])


def _tc_head(h, p0, p1, batch3d, w1, b1, w2, b2):
    full = lambda r, c: pl.BlockSpec((r, c), lambda i: (0, 0))
    return pl.pallas_call(
        _tc_head_body,
        grid=(_N // _BN,),
        in_specs=[
            pl.BlockSpec((_BN, _D), lambda i: (i, 0)),
            pl.BlockSpec((_BN, _D), lambda i: (i, 0)),
            pl.BlockSpec((_BN, _D), lambda i: (i, 0)),
            pl.BlockSpec((1, 1, _BN), lambda i: (i, 0, 0)),
            full(_D, _D), full(1, _D), full(_D, 1), full(1, 1),
        ],
        out_specs=[
            pl.BlockSpec((_BN, _D), lambda i: (i, 0)),
            pl.BlockSpec((_G, 1), lambda i: (0, 0)),
        ],
        out_shape=[
            jax.ShapeDtypeStruct((_N, _D), _f32),
            jax.ShapeDtypeStruct((_G, 1), _f32),
        ],
        scratch_shapes=[pltpu.VMEM((_G, _D), _f32)],
    )(h, p0, p1, batch3d, w1, b1, w2, b2)


def _layer(h, dst3d_g, src3d_g, dst3d_s, eab, zeros_sl, Wf, bf, Ws, bs):
    pi, pj = _sc_gather()(h, dst3d_g, src3d_g)
    wb = lambda t: t.astype(_bf16)
    msg = _tc_msg(
        pi, pj, eab,
        wb(Wf[:_D]), wb(Wf[_D:2 * _D]), wb(Wf[2 * _D:]), bf.reshape(1, _D),
        wb(Ws[:_D]), wb(Ws[_D:2 * _D]), wb(Ws[2 * _D:]), bs.reshape(1, _D),
    )
    parts = _sc_scatter()(msg, dst3d_s, zeros_sl)
    return parts[0], parts[1]


def kernel(x, edge_index, edge_attr, batch, W_node, b_node, Wf1, bf1, Ws1, bs1,
           Wf2, bf2, Ws2, bs2, W_fc1, b_fc1, W_fc2, b_fc2):
    dsti = edge_index[1].astype(jnp.int32)
    srci = edge_index[0].astype(jnp.int32)
    dst3d_g = dsti.reshape(_NW, _NCH, _CH)
    src3d_g = srci.reshape(_NW, _NCH, _CH)
    dst3d_s = dsti.reshape(_NW, _SNCH, _SCH)
    batch3d = batch.astype(jnp.int32).reshape(_N // _BN, 1, _BN)
    zeros_sl = jnp.zeros((_OBR, _D), _f32)

    eab = edge_attr.astype(_bf16)
    h = _tc_transform(x, W_node, b_node.reshape(1, _D))
    p0, p1 = _layer(h, dst3d_g, src3d_g, dst3d_s, eab, zeros_sl,
                    Wf1, bf1, Ws1, bs1)
    h1 = _tc_update(h, p0, p1)
    q0, q1 = _layer(h1, dst3d_g, src3d_g, dst3d_s, eab, zeros_sl,
                    Wf2, bf2, Ws2, bs2)
    atom_embs, out = _tc_head(h1, q0, q1, batch3d,
                              W_fc1, b_fc1.reshape(1, _D),
                              W_fc2, b_fc2.reshape(1, 1))
    return (out, atom_embs)
